# baseline (speedup 1.0000x reference)
import functools

import jax
import jax.numpy as jnp
from jax import lax
from jax.experimental import pallas as pl
from jax.experimental.pallas import tpu as pltpu
from jax.experimental.pallas import tpu_sc as plsc

N = 50000
NP = 50176
ROWS = NP // 4
H = 32
E = 1_600_000
EP = 32 * 392 * 128
CHUNK_ROWS = 8
PART_ITERS = 392 // CHUNK_ROWS
HALF = NP // 2
TRASH = 128
ACC = HALF + TRASH
APS = ACC // 16
CPS = HALF // 16
ZROWS = APS // 8
NG = 64
PBLK = 1568
PAIRW = 2048
SEGW = 25 * PAIRW
BUFW = SEGW + PAIRW

_f32 = jnp.float32
_sc_mesh = plsc.VectorSubcoreMesh(core_axis_name="c", subcore_axis_name="s")
_sc_params = pltpu.CompilerParams(use_tc_tiling_on_sc=False,
                                 needs_layout_passes=False)


def _fill_rows(buf, nrows, vec16):
    def body(i, carry):
        buf[i, pl.ds(0, 16)] = vec16
        buf[i, pl.ds(16, 16)] = vec16
        return carry

    lax.fori_loop(0, nrows, body, None)


def _zero_accumulator(zbuf, agg_sh, s):
    _fill_rows(zbuf, ZROWS, jnp.zeros((16,), _f32))
    base = s * APS
    for b in range(8):
        pltpu.sync_copy(zbuf, agg_sh.at[pl.ds(base + b * ZROWS, ZROWS)])
    plsc.subcore_barrier()


def _copy_out(agg_sh, out_hbm, c, s):
    plsc.subcore_barrier()
    pltpu.sync_copy(agg_sh.at[pl.ds(s * CPS, CPS)],
                    out_hbm.at[pl.ds(c * HALF + s * CPS, CPS)])


def _sc_part_body(src_hbm, dst_hbm, psrc, pdst, cnts,
                  srcb, dstb, bsrc, bdst, cntb, isem0, isem1, fsem):
    c = lax.axis_index("c")
    s = lax.axis_index("s")
    w = c * 16 + s
    isems = (isem0, isem1)
    trash16 = HALF + lax.iota(jnp.int32, 16)
    pad_src = jnp.full((16,), N, jnp.int32)

    for cc in range(2):
        base = cc * HALF

        def compact(b, off_v):
            for j in range(CHUNK_ROWS):
                for k in range(8):
                    sv = srcb[b, j, pl.ds(k * 16, 16)]
                    dv = dstb[b, j, pl.ds(k * 16, 16)]
                    t = dv - base
                    m = (t >= 0) & (t < HALF)
                    cs = plsc.cumsum(m.astype(jnp.int32))
                    idx = off_v + cs - 1
                    plsc.store_scatter(bsrc, [idx], sv, mask=m)
                    plsc.store_scatter(bdst, [idx], t, mask=m)
                    off_v = off_v + plsc.all_reduce_population_count(m)
            return off_v

        def fire_idx(t, b):
            pltpu.async_copy(src_hbm.at[w, pl.ds(t * CHUNK_ROWS, CHUNK_ROWS)],
                             srcb.at[b], isems[b])
            pltpu.async_copy(dst_hbm.at[w, pl.ds(t * CHUNK_ROWS, CHUNK_ROWS)],
                             dstb.at[b], isems[b])

        def drain_idx(b):
            pltpu.make_async_copy(src_hbm.at[w, pl.ds(0, CHUNK_ROWS)],
                                  srcb.at[b], isems[b]).wait()
            pltpu.make_async_copy(dst_hbm.at[w, pl.ds(0, CHUNK_ROWS)],
                                  dstb.at[b], isems[b]).wait()

        pltpu.sync_copy(src_hbm.at[w, pl.ds(0, CHUNK_ROWS)], srcb.at[0])
        pltpu.sync_copy(dst_hbm.at[w, pl.ds(0, CHUNK_ROWS)], dstb.at[0])
        fire_idx(1, 1)
        off_v = compact(0, jnp.zeros((16,), jnp.int32))

        def it_body(i, off_v):
            for half in range(2):
                b = (1, 0)[half]
                t = 2 * i + 1 + half
                drain_idx(b)

                @pl.when(t + 1 < PART_ITERS)
                def _():
                    fire_idx(t + 1, 1 - b)

                off_v = compact(b, off_v)
            return off_v

        off_v = lax.fori_loop(0, (PART_ITERS - 1) // 2, it_body, off_v)
        npair_v = jnp.maximum(1, (off_v + (PAIRW - 1)) // PAIRW)
        cntb[0, pl.ds(0, 16)] = off_v
        off = cntb[0, pl.ds(0, 16)][0]
        cntb[0, pl.ds(0, 16)] = npair_v
        npair = cntb[0, pl.ds(0, 16)][0]

        def pad_body(i, carry):
            bsrc[pl.ds(off + i * 16, 16)] = pad_src
            bdst[pl.ds(off + i * 16, 16)] = trash16
            return carry

        lax.fori_loop(0, PAIRW // 16, pad_body, None)

        def flush_body(f, carry):
            pltpu.async_copy(bsrc.at[pl.ds(f * PAIRW, PAIRW)],
                             psrc.at[cc, w, pl.ds(f * PAIRW, PAIRW)], fsem)
            pltpu.async_copy(bdst.at[pl.ds(f * PAIRW, PAIRW)],
                             pdst.at[cc, w, pl.ds(f * PAIRW, PAIRW)], fsem)
            return carry

        lax.fori_loop(0, npair, flush_body, None)

        def flushwait_body(f, carry):
            pltpu.make_async_copy(psrc.at[cc, w, pl.ds(0, PAIRW)],
                                  bsrc.at[pl.ds(0, PAIRW)], fsem).wait()
            pltpu.make_async_copy(pdst.at[cc, w, pl.ds(0, PAIRW)],
                                  bdst.at[pl.ds(0, PAIRW)], fsem).wait()
            return carry

        lax.fori_loop(0, npair, flushwait_body, None)
        pltpu.sync_copy(cntb, cnts.at[cc, w])


def _copy_idx(dstbuf, dstloc, b):
    for j in range(CHUNK_ROWS):
        for k in range(8):
            dstloc[b, j, pl.ds(k * 16, 16)] = dstbuf[b, j, pl.ds(k * 16, 16)]


def _sc_layer_body(y_hbm, psrc, pdst, cnts, out_hbm,
                   srcbuf, dstbuf, dstloc, rows, zbuf, cntb, agg_sh,
                   gsem0, gsem1, isem0, isem1, ssem0, ssem1):
    c = lax.axis_index("c")
    s = lax.axis_index("s")
    _zero_accumulator(zbuf, agg_sh, s)
    gsems = (gsem0, gsem1)
    isems = (isem0, isem1)
    ssems = (ssem0, ssem1)

    for seg in range(2):
        w = 2 * s + seg
        pltpu.sync_copy(cnts.at[c, w], cntb)
        npair = cntb[0, pl.ds(0, 16)][0]

        def fire_idx(t, b):
            for j in range(CHUNK_ROWS):
                pltpu.async_copy(
                    psrc.at[c, w, pl.ds(t * 1024 + j * 128, 128)],
                    srcbuf.at[b, j], isems[b])
                pltpu.async_copy(
                    pdst.at[c, w, pl.ds(t * 1024 + j * 128, 128)],
                    dstbuf.at[b, j], isems[b])

        def drain_idx(b):
            for j in range(CHUNK_ROWS):
                pltpu.make_async_copy(psrc.at[c, w, pl.ds(0, 128)],
                                      srcbuf.at[b, j], isems[b]).wait()
                pltpu.make_async_copy(pdst.at[c, w, pl.ds(0, 128)],
                                      dstbuf.at[b, j], isems[b]).wait()

        def fire_gathers(b):
            for j in range(CHUNK_ROWS):
                pltpu.async_copy(y_hbm.at[srcbuf.at[b, j]],
                                 rows.at[b, pl.ds(j * 128, 128)], gsems[b])

        def drain_gathers(b):
            pltpu.make_async_copy(y_hbm.at[pl.ds(0, CHUNK_ROWS * 128)],
                                  rows.at[b], gsems[b]).wait()

        def fire_scatters(b):
            for j in range(CHUNK_ROWS):
                pltpu.async_copy(rows.at[b, pl.ds(j * 128, 128)],
                                 agg_sh.at[dstloc.at[b, j]], ssems[b],
                                 add=True)

        def drain_scatters(b):
            pltpu.make_async_copy(y_hbm.at[pl.ds(0, CHUNK_ROWS * 128)],
                                  rows.at[b], ssems[b]).wait()

        for j in range(CHUNK_ROWS):
            pltpu.sync_copy(psrc.at[c, w, pl.ds(j * 128, 128)],
                            srcbuf.at[0, j])
            pltpu.sync_copy(pdst.at[c, w, pl.ds(j * 128, 128)],
                            dstbuf.at[0, j])
        fire_gathers(0)
        _copy_idx(dstbuf, dstloc, 0)
        fire_idx(1, 1)

        def it_body(i, carry):
            for half in range(2):
                b = (1, 0)[half]
                bb = 1 - b
                t = 2 * i + 1 + half
                if half == 0:
                    @pl.when(i > 0)
                    def _():
                        drain_scatters(b)
                else:
                    drain_scatters(b)
                drain_idx(b)
                fire_gathers(b)
                _copy_idx(dstbuf, dstloc, b)
                drain_gathers(bb)
                fire_idx(t + 1, bb)
                fire_scatters(bb)
            return carry

        lax.fori_loop(0, npair - 1, it_body, None)

        @pl.when(npair > 1)
        def _():
            drain_scatters(1)
        drain_idx(1)
        fire_gathers(1)
        _copy_idx(dstbuf, dstloc, 1)
        drain_gathers(0)
        fire_scatters(0)
        drain_gathers(1)
        fire_scatters(1)
        drain_scatters(0)
        drain_scatters(1)
    _copy_out(agg_sh, out_hbm, c, s)


def _sc_deg_body(pdst, cnts, out_hbm, dstbuf, dstloc, ones_rows, zbuf, cntb,
                 agg_sh, isem0, isem1, ssem0, ssem1):
    c = lax.axis_index("c")
    s = lax.axis_index("s")
    _fill_rows(ones_rows, CHUNK_ROWS * 128, jnp.ones((16,), _f32))
    _zero_accumulator(zbuf, agg_sh, s)
    isems = (isem0, isem1)
    ssems = (ssem0, ssem1)

    for seg in range(2):
        w = 2 * s + seg
        pltpu.sync_copy(cnts.at[c, w], cntb)
        npair = cntb[0, pl.ds(0, 16)][0]

        def fire_idx(t, b):
            for j in range(CHUNK_ROWS):
                pltpu.async_copy(
                    pdst.at[c, w, pl.ds(t * 1024 + j * 128, 128)],
                    dstbuf.at[b, j], isems[b])

        def drain_idx(b):
            for j in range(CHUNK_ROWS):
                pltpu.make_async_copy(pdst.at[c, w, pl.ds(0, 128)],
                                      dstbuf.at[b, j], isems[b]).wait()

        def fire_scatters(b):
            for j in range(CHUNK_ROWS):
                pltpu.async_copy(ones_rows.at[pl.ds(j * 128, 128)],
                                 agg_sh.at[dstloc.at[b, j]], ssems[b],
                                 add=True)

        def drain_scatters(b):
            pltpu.make_async_copy(out_hbm.at[pl.ds(0, CHUNK_ROWS * 128)],
                                  ones_rows, ssems[b]).wait()

        for j in range(CHUNK_ROWS):
            pltpu.sync_copy(pdst.at[c, w, pl.ds(j * 128, 128)],
                            dstbuf.at[0, j])
        _copy_idx(dstbuf, dstloc, 0)
        fire_idx(1, 1)
        fire_scatters(0)

        def it_body(i, carry):
            for half in range(2):
                b = (1, 0)[half]
                bb = 1 - b
                t = 2 * i + 1 + half
                if half == 0:
                    @pl.when(i > 0)
                    def _():
                        drain_scatters(b)
                else:
                    drain_scatters(b)
                drain_idx(b)
                _copy_idx(dstbuf, dstloc, b)
                fire_idx(t + 1, bb)
                fire_scatters(b)
            return carry

        lax.fori_loop(0, npair - 1, it_body, None)

        @pl.when(npair > 1)
        def _():
            drain_scatters(1)
        drain_idx(1)
        _copy_idx(dstbuf, dstloc, 1)
        fire_scatters(1)
        drain_scatters(0)
        drain_scatters(1)
    _copy_out(agg_sh, out_hbm, c, s)


_sc_part = pl.kernel(
    _sc_part_body,
    out_type=[
        jax.ShapeDtypeStruct((2, 32, SEGW), jnp.int32),
        jax.ShapeDtypeStruct((2, 32, SEGW), jnp.int32),
        jax.ShapeDtypeStruct((2, 32, 1, 16), jnp.int32),
    ],
    mesh=_sc_mesh,
    scratch_types=[
        pltpu.VMEM((2, CHUNK_ROWS, 128), jnp.int32),
        pltpu.VMEM((2, CHUNK_ROWS, 128), jnp.int32),
        pltpu.VMEM((BUFW,), jnp.int32),
        pltpu.VMEM((BUFW,), jnp.int32),
        pltpu.VMEM((1, 16), jnp.int32),
        pltpu.SemaphoreType.DMA,
        pltpu.SemaphoreType.DMA,
        pltpu.SemaphoreType.DMA,
    ],
    compiler_params=_sc_params,
)

_sc_layer = pl.kernel(
    _sc_layer_body,
    out_type=jax.ShapeDtypeStruct((NP, H), _f32),
    mesh=_sc_mesh,
    scratch_types=[
        pltpu.VMEM((2, CHUNK_ROWS, 128), jnp.int32),
        pltpu.VMEM((2, CHUNK_ROWS, 128), jnp.int32),
        pltpu.VMEM((2, CHUNK_ROWS, 128), jnp.int32),
        pltpu.VMEM((2, CHUNK_ROWS * 128, H), _f32),
        pltpu.VMEM((ZROWS, H), _f32),
        pltpu.VMEM((1, 16), jnp.int32),
        pltpu.VMEM_SHARED((ACC, H), _f32),
        pltpu.SemaphoreType.DMA,
        pltpu.SemaphoreType.DMA,
        pltpu.SemaphoreType.DMA,
        pltpu.SemaphoreType.DMA,
        pltpu.SemaphoreType.DMA,
        pltpu.SemaphoreType.DMA,
    ],
    compiler_params=_sc_params,
)

_sc_deg = pl.kernel(
    _sc_deg_body,
    out_type=jax.ShapeDtypeStruct((NP, H), _f32),
    mesh=_sc_mesh,
    scratch_types=[
        pltpu.VMEM((2, CHUNK_ROWS, 128), jnp.int32),
        pltpu.VMEM((2, CHUNK_ROWS, 128), jnp.int32),
        pltpu.VMEM((CHUNK_ROWS * 128, H), _f32),
        pltpu.VMEM((ZROWS, H), _f32),
        pltpu.VMEM((1, 16), jnp.int32),
        pltpu.VMEM_SHARED((ACC, H), _f32),
        pltpu.SemaphoreType.DMA,
        pltpu.SemaphoreType.DMA,
        pltpu.SemaphoreType.DMA,
        pltpu.SemaphoreType.DMA,
    ],
    compiler_params=_sc_params,
)

BR = 784
_TCGRID = ROWS // BR


def _tc_pro_body(x_ref, degp_ref, encw_ref, encb_ref, gw0_ref,
                 h_ref, y_ref, dinv_ref):
    g = pl.program_id(0)
    deg = degp_ref[...]
    r = lax.broadcasted_iota(jnp.int32, (BR, 128), 0)
    cc = lax.broadcasted_iota(jnp.int32, (BR, 128), 1)
    node = (g * BR + r) * 4 + cc // 32
    dinv = jnp.where(node < N, lax.rsqrt(deg + 1.0), 0.0)
    h0 = jax.nn.relu(
        jnp.dot(x_ref[...], encw_ref[...], preferred_element_type=_f32)
        + encb_ref[...])
    y_ref[...] = dinv * jnp.dot(h0, gw0_ref[...], preferred_element_type=_f32)
    h_ref[...] = h0
    dinv_ref[...] = dinv


def _tc_layer_body(last, h_ref, y_ref, aggp_ref, dinv_ref, b_ref, sw_ref,
                   gwn_ref, h_out, y_out=None):
    dinv = dinv_ref[...]
    agg = dinv * (aggp_ref[...] + y_ref[...]) + b_ref[...]
    t = jax.nn.relu(agg)
    h_new = h_ref[...] + jnp.dot(t, sw_ref[...], preferred_element_type=_f32)
    h_out[...] = h_new
    if not last:
        y_out[...] = dinv * jnp.dot(h_new, gwn_ref[...],
                                    preferred_element_type=_f32)


def _tc_epi_body(h_ref, b3_ref, w1_ref, b1_ref, w2_ref, b2_ref,
                 out_ref, acc_ref):
    g = pl.program_id(0)

    @pl.when(g == 0)
    def _():
        acc_ref[...] = jnp.zeros_like(acc_ref)

    bt = b3_ref[0]
    oh_t = (lax.broadcasted_iota(jnp.int32, (NG, PBLK), 0)
            == jnp.broadcast_to(bt, (NG, PBLK))).astype(_f32)
    haug = jnp.concatenate(
        [h_ref[...], jnp.ones((PBLK, 1), _f32)], axis=1)
    acc_ref[...] += jnp.dot(oh_t, haug, preferred_element_type=_f32)

    @pl.when(g == NP // PBLK - 1)
    def _():
        acc = acc_ref[...]
        pooled = acc[:, 0:H] / jnp.maximum(acc[:, H:H + 1], 1.0)
        hid = jax.nn.relu(
            jnp.dot(pooled, w1_ref[...], preferred_element_type=_f32)
            + b1_ref[...])
        out_ref[...] = (jnp.dot(hid, w2_ref[...], preferred_element_type=_f32)
                        + b2_ref[...])


def _full(shape):
    return pl.BlockSpec(shape, lambda g: (0,) * len(shape))


_tc_pro = pl.pallas_call(
    _tc_pro_body,
    grid=(_TCGRID,),
    in_specs=[
        pl.BlockSpec((BR, 16), lambda g: (g, 0)),
        pl.BlockSpec((BR, 128), lambda g: (g, 0)),
        _full((16, 128)),
        _full((1, 128)),
        _full((128, 128)),
    ],
    out_specs=[pl.BlockSpec((BR, 128), lambda g: (g, 0))] * 3,
    out_shape=[jax.ShapeDtypeStruct((ROWS, 128), _f32)] * 3,
)

_layer_in_specs = [
    pl.BlockSpec((BR, 128), lambda g: (g, 0)),
    pl.BlockSpec((BR, 128), lambda g: (g, 0)),
    pl.BlockSpec((BR, 128), lambda g: (g, 0)),
    pl.BlockSpec((BR, 128), lambda g: (g, 0)),
    _full((1, 128)),
    _full((128, 128)),
    _full((128, 128)),
]

_tc_layer = pl.pallas_call(
    functools.partial(_tc_layer_body, False),
    grid=(_TCGRID,),
    in_specs=_layer_in_specs,
    out_specs=[pl.BlockSpec((BR, 128), lambda g: (g, 0))] * 2,
    out_shape=[jax.ShapeDtypeStruct((ROWS, 128), _f32)] * 2,
)

_tc_layer_last = pl.pallas_call(
    functools.partial(_tc_layer_body, True),
    grid=(_TCGRID,),
    in_specs=_layer_in_specs,
    out_specs=pl.BlockSpec((BR, 128), lambda g: (g, 0)),
    out_shape=jax.ShapeDtypeStruct((ROWS, 128), _f32),
)

_tc_epi = pl.pallas_call(
    _tc_epi_body,
    grid=(NP // PBLK,),
    in_specs=[
        pl.BlockSpec((PBLK, H), lambda g: (g, 0)),
        pl.BlockSpec((1, 1, PBLK), lambda g: (g, 0, 0)),
        _full((H, 64)),
        _full((1, 64)),
        _full((64, 4)),
        _full((1, 4)),
    ],
    out_specs=_full((NG, 4)),
    out_shape=jax.ShapeDtypeStruct((NG, 4), _f32),
    scratch_shapes=[pltpu.VMEM((NG, H + 1), _f32)],
)


def kernel(x, edge_index, batch, enc_W, enc_b, gcn_W, gcn_b, symp_W,
           dec_W1, dec_b1, dec_W2, dec_b2):
    src = edge_index[0].astype(jnp.int32)
    dst = edge_index[1].astype(jnp.int32)
    epad = EP - E
    src3 = jnp.concatenate([src, jnp.full((epad,), N, jnp.int32)]
                           ).reshape(32, 392, 128)
    dst3 = jnp.concatenate([dst, jnp.full((epad,), N, jnp.int32)]
                           ).reshape(32, 392, 128)
    xp = jnp.pad(x.astype(_f32), ((0, NP - N), (0, 0))).reshape(ROWS, 16)
    b3 = jnp.pad(batch.astype(jnp.int32), (0, NP - N),
                 constant_values=NG).reshape(NP // PBLK, 1, PBLK)

    eye4 = jnp.eye(4, dtype=_f32)
    enc_bd = jnp.einsum("ab,ij->aibj", eye4,
                        enc_W.astype(_f32)).reshape(16, 128)
    gcn_bd = jnp.einsum("ab,lij->laibj", eye4,
                        gcn_W.astype(_f32)).reshape(5, 128, 128)
    symp_bd = jnp.einsum("ab,lij->laibj", eye4,
                         symp_W.astype(_f32)).reshape(5, 128, 128)
    enc_b4 = jnp.tile(enc_b.astype(_f32), 4).reshape(1, 128)
    gcn_b4 = jnp.tile(gcn_b.astype(_f32), (1, 4)).reshape(5, 1, 128)

    psrc, pdst, cnts = _sc_part(src3, dst3)
    deg_p = _sc_deg(pdst, cnts).reshape(ROWS, 128)
    h, y, dinv = _tc_pro(xp, deg_p, enc_bd, enc_b4, gcn_bd[0])
    for i in range(5):
        agg_p = _sc_layer(y.reshape(NP, H), psrc, pdst,
                          cnts).reshape(ROWS, 128)
        if i < 4:
            h, y = _tc_layer(h, y, agg_p, dinv, gcn_b4[i], symp_bd[i],
                             gcn_bd[i + 1])
        else:
            h = _tc_layer_last(h, y, agg_p, dinv, gcn_b4[i], symp_bd[i],
                               gcn_bd[0])
    return _tc_epi(h.reshape(NP, H), b3, dec_W1.astype(_f32),
                   dec_b1.astype(_f32).reshape(1, 64), dec_W2.astype(_f32),
                   dec_b2.astype(_f32).reshape(1, 4))


# speedup vs baseline: 1.0080x; 1.0080x over previous
import functools

import jax
import jax.numpy as jnp
from jax import lax
from jax.experimental import pallas as pl
from jax.experimental.pallas import tpu as pltpu
from jax.experimental.pallas import tpu_sc as plsc

N = 50000
NP = 50176
ROWS = NP // 4
H = 32
E = 1_600_000
EP = 32 * 392 * 128
CHUNK_ROWS = 8
PART_ITERS = 392 // CHUNK_ROWS
HALF = NP // 2
TRASH = 128
ACC = HALF + TRASH
APS = ACC // 16
CPS = HALF // 16
ZROWS = APS // 8
NG = 64
PBLK = 1568
PAIRW = 2048
SEGW = 25 * PAIRW
BUFW = SEGW + PAIRW

_f32 = jnp.float32
_sc_mesh = plsc.VectorSubcoreMesh(core_axis_name="c", subcore_axis_name="s")
_sc_params = pltpu.CompilerParams(use_tc_tiling_on_sc=False,
                                 needs_layout_passes=False)


def _fill_rows(buf, nrows, vec16):
    def body(i, carry):
        buf[i, pl.ds(0, 16)] = vec16
        buf[i, pl.ds(16, 16)] = vec16
        return carry

    lax.fori_loop(0, nrows, body, None)


def _zero_accumulator(zbuf, agg_sh, s):
    _fill_rows(zbuf, ZROWS, jnp.zeros((16,), _f32))
    base = s * APS
    for b in range(8):
        pltpu.sync_copy(zbuf, agg_sh.at[pl.ds(base + b * ZROWS, ZROWS)])
    plsc.subcore_barrier()


def _copy_out(agg_sh, out_hbm, c, s):
    plsc.subcore_barrier()
    pltpu.sync_copy(agg_sh.at[pl.ds(s * CPS, CPS)],
                    out_hbm.at[pl.ds(c * HALF + s * CPS, CPS)])


def _sc_part_body(src_hbm, dst_hbm, psrc, pdst, cnts,
                  srcb, dstb, bsrc, bdst, cntb, isem0, isem1, fsem):
    c = lax.axis_index("c")
    s = lax.axis_index("s")
    w = c * 16 + s
    isems = (isem0, isem1)
    trash16 = HALF + lax.iota(jnp.int32, 16)
    pad_src = jnp.full((16,), N, jnp.int32)

    for cc in range(2):
        base = cc * HALF

        def compact(b, off_v):
            for j in range(CHUNK_ROWS):
                for k in range(8):
                    sv = srcb[b, j, pl.ds(k * 16, 16)]
                    dv = dstb[b, j, pl.ds(k * 16, 16)]
                    t = dv - base
                    m = (t >= 0) & (t < HALF)
                    cs = plsc.cumsum(m.astype(jnp.int32))
                    idx = off_v + cs - 1
                    plsc.store_scatter(bsrc, [idx], sv, mask=m)
                    plsc.store_scatter(bdst, [idx], t, mask=m)
                    off_v = off_v + plsc.all_reduce_population_count(m)
            return off_v

        def fire_idx(t, b):
            pltpu.async_copy(src_hbm.at[w, pl.ds(t * CHUNK_ROWS, CHUNK_ROWS)],
                             srcb.at[b], isems[b])
            pltpu.async_copy(dst_hbm.at[w, pl.ds(t * CHUNK_ROWS, CHUNK_ROWS)],
                             dstb.at[b], isems[b])

        def drain_idx(b):
            pltpu.make_async_copy(src_hbm.at[w, pl.ds(0, CHUNK_ROWS)],
                                  srcb.at[b], isems[b]).wait()
            pltpu.make_async_copy(dst_hbm.at[w, pl.ds(0, CHUNK_ROWS)],
                                  dstb.at[b], isems[b]).wait()

        pltpu.sync_copy(src_hbm.at[w, pl.ds(0, CHUNK_ROWS)], srcb.at[0])
        pltpu.sync_copy(dst_hbm.at[w, pl.ds(0, CHUNK_ROWS)], dstb.at[0])
        fire_idx(1, 1)
        off_v = compact(0, jnp.zeros((16,), jnp.int32))

        def it_body(i, off_v):
            for half in range(2):
                b = (1, 0)[half]
                t = 2 * i + 1 + half
                drain_idx(b)

                @pl.when(t + 1 < PART_ITERS)
                def _():
                    fire_idx(t + 1, 1 - b)

                off_v = compact(b, off_v)
            return off_v

        off_v = lax.fori_loop(0, (PART_ITERS - 1) // 2, it_body, off_v)
        npair_v = jnp.maximum(1, (off_v + (PAIRW - 1)) // PAIRW)
        cntb[0, pl.ds(0, 16)] = off_v
        off = cntb[0, pl.ds(0, 16)][0]
        cntb[0, pl.ds(0, 16)] = npair_v
        npair = cntb[0, pl.ds(0, 16)][0]

        def pad_body(i, carry):
            bsrc[pl.ds(off + i * 16, 16)] = pad_src
            bdst[pl.ds(off + i * 16, 16)] = trash16
            return carry

        lax.fori_loop(0, PAIRW // 16, pad_body, None)

        def flush_body(f, carry):
            pltpu.async_copy(bsrc.at[pl.ds(f * PAIRW, PAIRW)],
                             psrc.at[cc, w, pl.ds(f * PAIRW, PAIRW)], fsem)
            pltpu.async_copy(bdst.at[pl.ds(f * PAIRW, PAIRW)],
                             pdst.at[cc, w, pl.ds(f * PAIRW, PAIRW)], fsem)
            return carry

        lax.fori_loop(0, npair, flush_body, None)

        def flushwait_body(f, carry):
            pltpu.make_async_copy(psrc.at[cc, w, pl.ds(0, PAIRW)],
                                  bsrc.at[pl.ds(0, PAIRW)], fsem).wait()
            pltpu.make_async_copy(pdst.at[cc, w, pl.ds(0, PAIRW)],
                                  bdst.at[pl.ds(0, PAIRW)], fsem).wait()
            return carry

        lax.fori_loop(0, npair, flushwait_body, None)
        pltpu.sync_copy(cntb, cnts.at[cc, w])


def _copy_idx(dstbuf, dstloc, b):
    for j in range(CHUNK_ROWS):
        for k in range(8):
            dstloc[b, j, pl.ds(k * 16, 16)] = dstbuf[
                b, pl.ds(j * 128 + k * 16, 16)]


def _sc_layer_body(y_hbm, psrc, pdst, cnts, out_hbm,
                   srcbuf, dstbuf, dstloc, rows, zbuf, cntb, agg_sh,
                   gsem0, gsem1, isem0, isem1, ssem0, ssem1):
    c = lax.axis_index("c")
    s = lax.axis_index("s")
    _zero_accumulator(zbuf, agg_sh, s)
    gsems = (gsem0, gsem1)
    isems = (isem0, isem1)
    ssems = (ssem0, ssem1)

    for seg in range(2):
        w = 2 * s + seg
        pltpu.sync_copy(cnts.at[c, w], cntb)
        npair = cntb[0, pl.ds(0, 16)][0]

        def fire_idx(t, b):
            pltpu.async_copy(psrc.at[c, w, pl.ds(t * 1024, 1024)],
                             srcbuf.at[b], isems[b])
            pltpu.async_copy(pdst.at[c, w, pl.ds(t * 1024, 1024)],
                             dstbuf.at[b], isems[b])

        def drain_idx(b):
            pltpu.make_async_copy(psrc.at[c, w, pl.ds(0, 1024)],
                                  srcbuf.at[b], isems[b]).wait()
            pltpu.make_async_copy(pdst.at[c, w, pl.ds(0, 1024)],
                                  dstbuf.at[b], isems[b]).wait()

        def fire_gathers(b):
            for j in range(CHUNK_ROWS):
                pltpu.async_copy(y_hbm.at[srcbuf.at[b, pl.ds(j * 128, 128)]],
                                 rows.at[b, pl.ds(j * 128, 128)], gsems[b])

        def drain_gathers(b):
            pltpu.make_async_copy(y_hbm.at[pl.ds(0, CHUNK_ROWS * 128)],
                                  rows.at[b], gsems[b]).wait()

        def fire_scatters(b):
            for j in range(CHUNK_ROWS):
                pltpu.async_copy(rows.at[b, pl.ds(j * 128, 128)],
                                 agg_sh.at[dstloc.at[b, j]], ssems[b],
                                 add=True)

        def drain_scatters(b):
            pltpu.make_async_copy(y_hbm.at[pl.ds(0, CHUNK_ROWS * 128)],
                                  rows.at[b], ssems[b]).wait()

        pltpu.sync_copy(psrc.at[c, w, pl.ds(0, 1024)], srcbuf.at[0])
        pltpu.sync_copy(pdst.at[c, w, pl.ds(0, 1024)], dstbuf.at[0])
        fire_gathers(0)
        _copy_idx(dstbuf, dstloc, 0)
        fire_idx(1, 1)

        def it_body(i, carry):
            for half in range(2):
                b = (1, 0)[half]
                bb = 1 - b
                t = 2 * i + 1 + half
                if half == 0:
                    @pl.when(i > 0)
                    def _():
                        drain_scatters(b)
                else:
                    drain_scatters(b)
                drain_idx(b)
                fire_gathers(b)
                _copy_idx(dstbuf, dstloc, b)
                drain_gathers(bb)
                fire_idx(t + 1, bb)
                fire_scatters(bb)
            return carry

        lax.fori_loop(0, npair - 1, it_body, None)

        @pl.when(npair > 1)
        def _():
            drain_scatters(1)
        drain_idx(1)
        fire_gathers(1)
        _copy_idx(dstbuf, dstloc, 1)
        drain_gathers(0)
        fire_scatters(0)
        drain_gathers(1)
        fire_scatters(1)
        drain_scatters(0)
        drain_scatters(1)
    _copy_out(agg_sh, out_hbm, c, s)


def _sc_deg_body(pdst, cnts, out_hbm, dstbuf, dstloc, ones_rows, zbuf, cntb,
                 agg_sh, isem0, isem1, ssem0, ssem1):
    c = lax.axis_index("c")
    s = lax.axis_index("s")
    _fill_rows(ones_rows, CHUNK_ROWS * 128, jnp.ones((16,), _f32))
    _zero_accumulator(zbuf, agg_sh, s)
    isems = (isem0, isem1)
    ssems = (ssem0, ssem1)

    for seg in range(2):
        w = 2 * s + seg
        pltpu.sync_copy(cnts.at[c, w], cntb)
        npair = cntb[0, pl.ds(0, 16)][0]

        def fire_idx(t, b):
            pltpu.async_copy(pdst.at[c, w, pl.ds(t * 1024, 1024)],
                             dstbuf.at[b], isems[b])

        def drain_idx(b):
            pltpu.make_async_copy(pdst.at[c, w, pl.ds(0, 1024)],
                                  dstbuf.at[b], isems[b]).wait()

        def fire_scatters(b):
            for j in range(CHUNK_ROWS):
                pltpu.async_copy(ones_rows.at[pl.ds(j * 128, 128)],
                                 agg_sh.at[dstloc.at[b, j]], ssems[b],
                                 add=True)

        def drain_scatters(b):
            pltpu.make_async_copy(out_hbm.at[pl.ds(0, CHUNK_ROWS * 128)],
                                  ones_rows, ssems[b]).wait()

        pltpu.sync_copy(pdst.at[c, w, pl.ds(0, 1024)], dstbuf.at[0])
        _copy_idx(dstbuf, dstloc, 0)
        fire_idx(1, 1)
        fire_scatters(0)

        def it_body(i, carry):
            for half in range(2):
                b = (1, 0)[half]
                bb = 1 - b
                t = 2 * i + 1 + half
                if half == 0:
                    @pl.when(i > 0)
                    def _():
                        drain_scatters(b)
                else:
                    drain_scatters(b)
                drain_idx(b)
                _copy_idx(dstbuf, dstloc, b)
                fire_idx(t + 1, bb)
                fire_scatters(b)
            return carry

        lax.fori_loop(0, npair - 1, it_body, None)

        @pl.when(npair > 1)
        def _():
            drain_scatters(1)
        drain_idx(1)
        _copy_idx(dstbuf, dstloc, 1)
        fire_scatters(1)
        drain_scatters(0)
        drain_scatters(1)
    _copy_out(agg_sh, out_hbm, c, s)


_sc_part = pl.kernel(
    _sc_part_body,
    out_type=[
        jax.ShapeDtypeStruct((2, 32, SEGW), jnp.int32),
        jax.ShapeDtypeStruct((2, 32, SEGW), jnp.int32),
        jax.ShapeDtypeStruct((2, 32, 1, 16), jnp.int32),
    ],
    mesh=_sc_mesh,
    scratch_types=[
        pltpu.VMEM((2, CHUNK_ROWS, 128), jnp.int32),
        pltpu.VMEM((2, CHUNK_ROWS, 128), jnp.int32),
        pltpu.VMEM((BUFW,), jnp.int32),
        pltpu.VMEM((BUFW,), jnp.int32),
        pltpu.VMEM((1, 16), jnp.int32),
        pltpu.SemaphoreType.DMA,
        pltpu.SemaphoreType.DMA,
        pltpu.SemaphoreType.DMA,
    ],
    compiler_params=_sc_params,
)

_sc_layer = pl.kernel(
    _sc_layer_body,
    out_type=jax.ShapeDtypeStruct((NP, H), _f32),
    mesh=_sc_mesh,
    scratch_types=[
        pltpu.VMEM((2, CHUNK_ROWS * 128), jnp.int32),
        pltpu.VMEM((2, CHUNK_ROWS * 128), jnp.int32),
        pltpu.VMEM((2, CHUNK_ROWS, 128), jnp.int32),
        pltpu.VMEM((2, CHUNK_ROWS * 128, H), _f32),
        pltpu.VMEM((ZROWS, H), _f32),
        pltpu.VMEM((1, 16), jnp.int32),
        pltpu.VMEM_SHARED((ACC, H), _f32),
        pltpu.SemaphoreType.DMA,
        pltpu.SemaphoreType.DMA,
        pltpu.SemaphoreType.DMA,
        pltpu.SemaphoreType.DMA,
        pltpu.SemaphoreType.DMA,
        pltpu.SemaphoreType.DMA,
    ],
    compiler_params=_sc_params,
)

_sc_deg = pl.kernel(
    _sc_deg_body,
    out_type=jax.ShapeDtypeStruct((NP, H), _f32),
    mesh=_sc_mesh,
    scratch_types=[
        pltpu.VMEM((2, CHUNK_ROWS * 128), jnp.int32),
        pltpu.VMEM((2, CHUNK_ROWS, 128), jnp.int32),
        pltpu.VMEM((CHUNK_ROWS * 128, H), _f32),
        pltpu.VMEM((ZROWS, H), _f32),
        pltpu.VMEM((1, 16), jnp.int32),
        pltpu.VMEM_SHARED((ACC, H), _f32),
        pltpu.SemaphoreType.DMA,
        pltpu.SemaphoreType.DMA,
        pltpu.SemaphoreType.DMA,
        pltpu.SemaphoreType.DMA,
    ],
    compiler_params=_sc_params,
)

BR = 784
_TCGRID = ROWS // BR


def _tc_pro_body(x_ref, degp_ref, encw_ref, encb_ref, gw0_ref,
                 h_ref, y_ref, dinv_ref):
    g = pl.program_id(0)
    deg = degp_ref[...]
    r = lax.broadcasted_iota(jnp.int32, (BR, 128), 0)
    cc = lax.broadcasted_iota(jnp.int32, (BR, 128), 1)
    node = (g * BR + r) * 4 + cc // 32
    dinv = jnp.where(node < N, lax.rsqrt(deg + 1.0), 0.0)
    h0 = jax.nn.relu(
        jnp.dot(x_ref[...], encw_ref[...], preferred_element_type=_f32)
        + encb_ref[...])
    y_ref[...] = dinv * jnp.dot(h0, gw0_ref[...], preferred_element_type=_f32)
    h_ref[...] = h0
    dinv_ref[...] = dinv


def _tc_layer_body(last, h_ref, y_ref, aggp_ref, dinv_ref, b_ref, sw_ref,
                   gwn_ref, h_out, y_out=None):
    dinv = dinv_ref[...]
    agg = dinv * (aggp_ref[...] + y_ref[...]) + b_ref[...]
    t = jax.nn.relu(agg)
    h_new = h_ref[...] + jnp.dot(t, sw_ref[...], preferred_element_type=_f32)
    h_out[...] = h_new
    if not last:
        y_out[...] = dinv * jnp.dot(h_new, gwn_ref[...],
                                    preferred_element_type=_f32)


def _tc_epi_body(h_ref, b3_ref, w1_ref, b1_ref, w2_ref, b2_ref,
                 out_ref, acc_ref):
    g = pl.program_id(0)

    @pl.when(g == 0)
    def _():
        acc_ref[...] = jnp.zeros_like(acc_ref)

    bt = b3_ref[0]
    oh_t = (lax.broadcasted_iota(jnp.int32, (NG, PBLK), 0)
            == jnp.broadcast_to(bt, (NG, PBLK))).astype(_f32)
    haug = jnp.concatenate(
        [h_ref[...], jnp.ones((PBLK, 1), _f32)], axis=1)
    acc_ref[...] += jnp.dot(oh_t, haug, preferred_element_type=_f32)

    @pl.when(g == NP // PBLK - 1)
    def _():
        acc = acc_ref[...]
        pooled = acc[:, 0:H] / jnp.maximum(acc[:, H:H + 1], 1.0)
        hid = jax.nn.relu(
            jnp.dot(pooled, w1_ref[...], preferred_element_type=_f32)
            + b1_ref[...])
        out_ref[...] = (jnp.dot(hid, w2_ref[...], preferred_element_type=_f32)
                        + b2_ref[...])


def _full(shape):
    return pl.BlockSpec(shape, lambda g: (0,) * len(shape))


_tc_pro = pl.pallas_call(
    _tc_pro_body,
    grid=(_TCGRID,),
    in_specs=[
        pl.BlockSpec((BR, 16), lambda g: (g, 0)),
        pl.BlockSpec((BR, 128), lambda g: (g, 0)),
        _full((16, 128)),
        _full((1, 128)),
        _full((128, 128)),
    ],
    out_specs=[pl.BlockSpec((BR, 128), lambda g: (g, 0))] * 3,
    out_shape=[jax.ShapeDtypeStruct((ROWS, 128), _f32)] * 3,
)

_layer_in_specs = [
    pl.BlockSpec((BR, 128), lambda g: (g, 0)),
    pl.BlockSpec((BR, 128), lambda g: (g, 0)),
    pl.BlockSpec((BR, 128), lambda g: (g, 0)),
    pl.BlockSpec((BR, 128), lambda g: (g, 0)),
    _full((1, 128)),
    _full((128, 128)),
    _full((128, 128)),
]

_tc_layer = pl.pallas_call(
    functools.partial(_tc_layer_body, False),
    grid=(_TCGRID,),
    in_specs=_layer_in_specs,
    out_specs=[pl.BlockSpec((BR, 128), lambda g: (g, 0))] * 2,
    out_shape=[jax.ShapeDtypeStruct((ROWS, 128), _f32)] * 2,
)

_tc_layer_last = pl.pallas_call(
    functools.partial(_tc_layer_body, True),
    grid=(_TCGRID,),
    in_specs=_layer_in_specs,
    out_specs=pl.BlockSpec((BR, 128), lambda g: (g, 0)),
    out_shape=jax.ShapeDtypeStruct((ROWS, 128), _f32),
)

_tc_epi = pl.pallas_call(
    _tc_epi_body,
    grid=(NP // PBLK,),
    in_specs=[
        pl.BlockSpec((PBLK, H), lambda g: (g, 0)),
        pl.BlockSpec((1, 1, PBLK), lambda g: (g, 0, 0)),
        _full((H, 64)),
        _full((1, 64)),
        _full((64, 4)),
        _full((1, 4)),
    ],
    out_specs=_full((NG, 4)),
    out_shape=jax.ShapeDtypeStruct((NG, 4), _f32),
    scratch_shapes=[pltpu.VMEM((NG, H + 1), _f32)],
)


def kernel(x, edge_index, batch, enc_W, enc_b, gcn_W, gcn_b, symp_W,
           dec_W1, dec_b1, dec_W2, dec_b2):
    src = edge_index[0].astype(jnp.int32)
    dst = edge_index[1].astype(jnp.int32)
    epad = EP - E
    src3 = jnp.concatenate([src, jnp.full((epad,), N, jnp.int32)]
                           ).reshape(32, 392, 128)
    dst3 = jnp.concatenate([dst, jnp.full((epad,), N, jnp.int32)]
                           ).reshape(32, 392, 128)
    xp = jnp.pad(x.astype(_f32), ((0, NP - N), (0, 0))).reshape(ROWS, 16)
    b3 = jnp.pad(batch.astype(jnp.int32), (0, NP - N),
                 constant_values=NG).reshape(NP // PBLK, 1, PBLK)

    eye4 = jnp.eye(4, dtype=_f32)
    enc_bd = jnp.einsum("ab,ij->aibj", eye4,
                        enc_W.astype(_f32)).reshape(16, 128)
    gcn_bd = jnp.einsum("ab,lij->laibj", eye4,
                        gcn_W.astype(_f32)).reshape(5, 128, 128)
    symp_bd = jnp.einsum("ab,lij->laibj", eye4,
                         symp_W.astype(_f32)).reshape(5, 128, 128)
    enc_b4 = jnp.tile(enc_b.astype(_f32), 4).reshape(1, 128)
    gcn_b4 = jnp.tile(gcn_b.astype(_f32), (1, 4)).reshape(5, 1, 128)

    psrc, pdst, cnts = _sc_part(src3, dst3)
    deg_p = _sc_deg(pdst, cnts).reshape(ROWS, 128)
    h, y, dinv = _tc_pro(xp, deg_p, enc_bd, enc_b4, gcn_bd[0])
    for i in range(5):
        agg_p = _sc_layer(y.reshape(NP, H), psrc, pdst,
                          cnts).reshape(ROWS, 128)
        if i < 4:
            h, y = _tc_layer(h, y, agg_p, dinv, gcn_b4[i], symp_bd[i],
                             gcn_bd[i + 1])
        else:
            h = _tc_layer_last(h, y, agg_p, dinv, gcn_b4[i], symp_bd[i],
                               gcn_bd[0])
    return _tc_epi(h.reshape(NP, H), b3, dec_W1.astype(_f32),
                   dec_b1.astype(_f32).reshape(1, 64), dec_W2.astype(_f32),
                   dec_b2.astype(_f32).reshape(1, 4))


# speedup vs baseline: 2.9295x; 2.9062x over previous
import functools

import jax
import jax.numpy as jnp
from jax import lax
from jax.experimental import pallas as pl
from jax.experimental.pallas import tpu as pltpu
from jax.experimental.pallas import tpu_sc as plsc

N = 50000
NP = 50176
ROWS = NP // 4
H = 32
E = 1_600_000
EP = 16 * 784 * 128
CHUNK_ROWS = 8
N_ITERS = 784 // CHUNK_ROWS
HALF = NP // 2
TRASH = 128
ACC = HALF + TRASH
APS = ACC // 16
CPS = HALF // 16
ZROWS = APS // 8
NG = 64
PBLK = 1568

_f32 = jnp.float32
_sc_mesh = plsc.VectorSubcoreMesh(core_axis_name="c", subcore_axis_name="s")
_sc_params = pltpu.CompilerParams(use_tc_tiling_on_sc=False)


def _fill_rows(buf, nrows, vec16):
    def body(i, carry):
        buf[i, pl.ds(0, 16)] = vec16
        buf[i, pl.ds(16, 16)] = vec16
        return carry

    lax.fori_loop(0, nrows, body, None)


def _zero_accumulator(zbuf, agg_sh, s):
    _fill_rows(zbuf, ZROWS, jnp.zeros((16,), _f32))
    base = s * APS
    for b in range(8):
        pltpu.sync_copy(zbuf, agg_sh.at[pl.ds(base + b * ZROWS, ZROWS)])
    plsc.subcore_barrier()


def _copy_out(agg_sh, out_hbm, c, s):
    plsc.subcore_barrier()
    pltpu.sync_copy(agg_sh.at[pl.ds(s * CPS, CPS)],
                    out_hbm.at[pl.ds(c * HALF + s * CPS, CPS)])


def _remap_dst(dstbuf, dstloc, b, c):
    base = c * HALF
    for j in range(CHUNK_ROWS):
        for k in range(8):
            v = dstbuf[b, j, pl.ds(k * 16, 16)]
            t = v - base
            ok = (t >= 0) & (t < HALF)
            dstloc[b, j, pl.ds(k * 16, 16)] = jnp.where(
                ok, t, HALF + (v & (TRASH - 1)))


def _sc_layer_body(y_hbm, src_hbm, dst_hbm, out_hbm,
                   srcbuf, dstbuf, dstloc, rows, zbuf, agg_sh,
                   gsem0, gsem1, isem0, isem1, ssem0, ssem1):
    c = lax.axis_index("c")
    s = lax.axis_index("s")
    _zero_accumulator(zbuf, agg_sh, s)
    gsems = (gsem0, gsem1)
    isems = (isem0, isem1)
    ssems = (ssem0, ssem1)

    def fire_idx(t, b, sem):
        pltpu.async_copy(src_hbm.at[s, pl.ds(t * CHUNK_ROWS, CHUNK_ROWS)],
                         srcbuf.at[b], sem)
        pltpu.async_copy(dst_hbm.at[s, pl.ds(t * CHUNK_ROWS, CHUNK_ROWS)],
                         dstbuf.at[b], sem)

    def drain_idx(b):
        pltpu.make_async_copy(src_hbm.at[s, pl.ds(0, CHUNK_ROWS)],
                              srcbuf.at[b], isems[b]).wait()
        pltpu.make_async_copy(dst_hbm.at[s, pl.ds(0, CHUNK_ROWS)],
                              dstbuf.at[b], isems[b]).wait()

    def fire_gathers(b):
        for j in range(CHUNK_ROWS):
            pltpu.async_copy(y_hbm.at[srcbuf.at[b, j]],
                             rows.at[b, pl.ds(j * 128, 128)], gsems[b])

    def drain_gathers(b):
        pltpu.make_async_copy(y_hbm.at[pl.ds(0, CHUNK_ROWS * 128)],
                              rows.at[b], gsems[b]).wait()

    def fire_scatters(b):
        for j in range(CHUNK_ROWS):
            pltpu.async_copy(rows.at[b, pl.ds(j * 128, 128)],
                             agg_sh.at[dstloc.at[b, j]], ssems[b], add=True)

    def drain_scatters(b):
        pltpu.make_async_copy(y_hbm.at[pl.ds(0, CHUNK_ROWS * 128)],
                              rows.at[b], ssems[b]).wait()

    pltpu.sync_copy(src_hbm.at[s, pl.ds(0, CHUNK_ROWS)], srcbuf.at[0])
    pltpu.sync_copy(dst_hbm.at[s, pl.ds(0, CHUNK_ROWS)], dstbuf.at[0])
    fire_gathers(0)
    fire_idx(1, 1, isems[1])
    _remap_dst(dstbuf, dstloc, 0, c)

    def it_body(i, carry):
        for half in range(2):
            b = (1, 0)[half]
            bb = 1 - b
            t = 2 * i + 1 + half
            if half == 0:
                @pl.when(i > 0)
                def _():
                    drain_scatters(b)
            else:
                drain_scatters(b)
            drain_idx(b)
            fire_gathers(b)
            drain_gathers(bb)
            fire_idx(t + 1, bb, isems[bb])
            _remap_dst(dstbuf, dstloc, b, c)
            fire_scatters(bb)
        return carry

    lax.fori_loop(0, (N_ITERS - 2) // 2, it_body, None)

    drain_scatters(1)
    drain_idx(1)
    fire_gathers(1)
    drain_gathers(0)
    _remap_dst(dstbuf, dstloc, 1, c)
    fire_scatters(0)
    drain_gathers(1)
    fire_scatters(1)
    drain_scatters(0)
    drain_scatters(1)
    _copy_out(agg_sh, out_hbm, c, s)


def _sc_deg_body(dst_hbm, out_hbm, dstbuf, dstloc, ones_rows, zbuf, agg_sh,
                 isem0, isem1, ssem0, ssem1):
    c = lax.axis_index("c")
    s = lax.axis_index("s")
    _fill_rows(ones_rows, CHUNK_ROWS * 128, jnp.ones((16,), _f32))
    _zero_accumulator(zbuf, agg_sh, s)
    isems = (isem0, isem1)
    ssems = (ssem0, ssem1)

    def drain_idx(b):
        pltpu.make_async_copy(dst_hbm.at[s, pl.ds(0, CHUNK_ROWS)],
                              dstbuf.at[b], isems[b]).wait()

    def fire_scatters(b):
        for j in range(CHUNK_ROWS):
            pltpu.async_copy(ones_rows.at[pl.ds(j * 128, 128)],
                             agg_sh.at[dstloc.at[b, j]], ssems[b], add=True)

    def drain_scatters(b):
        pltpu.make_async_copy(out_hbm.at[pl.ds(0, CHUNK_ROWS * 128)],
                              ones_rows, ssems[b]).wait()

    pltpu.sync_copy(dst_hbm.at[s, pl.ds(0, CHUNK_ROWS)], dstbuf.at[0])
    pltpu.async_copy(dst_hbm.at[s, pl.ds(CHUNK_ROWS, CHUNK_ROWS)],
                     dstbuf.at[1], isems[1])
    _remap_dst(dstbuf, dstloc, 0, c)
    fire_scatters(0)

    def it_body(i, carry):
        for half in range(2):
            b = (1, 0)[half]
            bb = 1 - b
            t = 2 * i + 1 + half
            drain_idx(b)
            pltpu.async_copy(
                dst_hbm.at[s, pl.ds((t + 1) * CHUNK_ROWS, CHUNK_ROWS)],
                dstbuf.at[bb], isems[bb])
            if half == 0:
                @pl.when(i > 0)
                def _():
                    drain_scatters(b)
            else:
                drain_scatters(b)
            _remap_dst(dstbuf, dstloc, b, c)
            fire_scatters(b)
        return carry

    lax.fori_loop(0, (N_ITERS - 2) // 2, it_body, None)

    drain_idx(1)
    drain_scatters(1)
    _remap_dst(dstbuf, dstloc, 1, c)
    fire_scatters(1)
    drain_scatters(0)
    drain_scatters(1)
    _copy_out(agg_sh, out_hbm, c, s)


_sc_layer = pl.kernel(
    _sc_layer_body,
    out_type=jax.ShapeDtypeStruct((NP, H), _f32),
    mesh=_sc_mesh,
    scratch_types=[
        pltpu.VMEM((2, CHUNK_ROWS, 128), jnp.int32),
        pltpu.VMEM((2, CHUNK_ROWS, 128), jnp.int32),
        pltpu.VMEM((2, CHUNK_ROWS, 128), jnp.int32),
        pltpu.VMEM((2, CHUNK_ROWS * 128, H), _f32),
        pltpu.VMEM((ZROWS, H), _f32),
        pltpu.VMEM_SHARED((ACC, H), _f32),
        pltpu.SemaphoreType.DMA,
        pltpu.SemaphoreType.DMA,
        pltpu.SemaphoreType.DMA,
        pltpu.SemaphoreType.DMA,
        pltpu.SemaphoreType.DMA,
        pltpu.SemaphoreType.DMA,
    ],
    compiler_params=_sc_params,
)

_sc_deg = pl.kernel(
    _sc_deg_body,
    out_type=jax.ShapeDtypeStruct((NP, H), _f32),
    mesh=_sc_mesh,
    scratch_types=[
        pltpu.VMEM((2, CHUNK_ROWS, 128), jnp.int32),
        pltpu.VMEM((2, CHUNK_ROWS, 128), jnp.int32),
        pltpu.VMEM((CHUNK_ROWS * 128, H), _f32),
        pltpu.VMEM((ZROWS, H), _f32),
        pltpu.VMEM_SHARED((ACC, H), _f32),
        pltpu.SemaphoreType.DMA,
        pltpu.SemaphoreType.DMA,
        pltpu.SemaphoreType.DMA,
        pltpu.SemaphoreType.DMA,
    ],
    compiler_params=_sc_params,
)

BR = 784
_TCGRID = ROWS // BR


def _tc_pro_body(x_ref, degp_ref, encw_ref, encb_ref, gw0_ref,
                 h_ref, y_ref, dinv_ref):
    g = pl.program_id(0)
    deg = degp_ref[...]
    r = lax.broadcasted_iota(jnp.int32, (BR, 128), 0)
    cc = lax.broadcasted_iota(jnp.int32, (BR, 128), 1)
    node = (g * BR + r) * 4 + cc // 32
    dinv = jnp.where(node < N, lax.rsqrt(deg + 1.0), 0.0)
    h0 = jax.nn.relu(
        jnp.dot(x_ref[...], encw_ref[...], preferred_element_type=_f32)
        + encb_ref[...])
    y_ref[...] = dinv * jnp.dot(h0, gw0_ref[...], preferred_element_type=_f32)
    h_ref[...] = h0
    dinv_ref[...] = dinv


def _tc_layer_body(last, h_ref, y_ref, aggp_ref, dinv_ref, b_ref, sw_ref,
                   gwn_ref, h_out, y_out=None):
    dinv = dinv_ref[...]
    agg = dinv * (aggp_ref[...] + y_ref[...]) + b_ref[...]
    t = jax.nn.relu(agg)
    h_new = h_ref[...] + jnp.dot(t, sw_ref[...], preferred_element_type=_f32)
    h_out[...] = h_new
    if not last:
        y_out[...] = dinv * jnp.dot(h_new, gwn_ref[...],
                                    preferred_element_type=_f32)


def _tc_epi_body(h_ref, b3_ref, w1_ref, b1_ref, w2_ref, b2_ref,
                 out_ref, acc_ref):
    g = pl.program_id(0)

    @pl.when(g == 0)
    def _():
        acc_ref[...] = jnp.zeros_like(acc_ref)

    bt = b3_ref[0]
    oh_t = (lax.broadcasted_iota(jnp.int32, (NG, PBLK), 0)
            == jnp.broadcast_to(bt, (NG, PBLK))).astype(_f32)
    haug = jnp.concatenate(
        [h_ref[...], jnp.ones((PBLK, 1), _f32)], axis=1)
    acc_ref[...] += jnp.dot(oh_t, haug, preferred_element_type=_f32)

    @pl.when(g == NP // PBLK - 1)
    def _():
        acc = acc_ref[...]
        pooled = acc[:, 0:H] / jnp.maximum(acc[:, H:H + 1], 1.0)
        hid = jax.nn.relu(
            jnp.dot(pooled, w1_ref[...], preferred_element_type=_f32)
            + b1_ref[...])
        out_ref[...] = (jnp.dot(hid, w2_ref[...], preferred_element_type=_f32)
                        + b2_ref[...])


def _full(shape):
    return pl.BlockSpec(shape, lambda g: (0,) * len(shape))


_tc_pro = pl.pallas_call(
    _tc_pro_body,
    grid=(_TCGRID,),
    in_specs=[
        pl.BlockSpec((BR, 16), lambda g: (g, 0)),
        pl.BlockSpec((BR, 128), lambda g: (g, 0)),
        _full((16, 128)),
        _full((1, 128)),
        _full((128, 128)),
    ],
    out_specs=[pl.BlockSpec((BR, 128), lambda g: (g, 0))] * 3,
    out_shape=[jax.ShapeDtypeStruct((ROWS, 128), _f32)] * 3,
)

_layer_in_specs = [
    pl.BlockSpec((BR, 128), lambda g: (g, 0)),
    pl.BlockSpec((BR, 128), lambda g: (g, 0)),
    pl.BlockSpec((BR, 128), lambda g: (g, 0)),
    pl.BlockSpec((BR, 128), lambda g: (g, 0)),
    _full((1, 128)),
    _full((128, 128)),
    _full((128, 128)),
]

_tc_layer = pl.pallas_call(
    functools.partial(_tc_layer_body, False),
    grid=(_TCGRID,),
    in_specs=_layer_in_specs,
    out_specs=[pl.BlockSpec((BR, 128), lambda g: (g, 0))] * 2,
    out_shape=[jax.ShapeDtypeStruct((ROWS, 128), _f32)] * 2,
)

_tc_layer_last = pl.pallas_call(
    functools.partial(_tc_layer_body, True),
    grid=(_TCGRID,),
    in_specs=_layer_in_specs,
    out_specs=pl.BlockSpec((BR, 128), lambda g: (g, 0)),
    out_shape=jax.ShapeDtypeStruct((ROWS, 128), _f32),
)

_tc_epi = pl.pallas_call(
    _tc_epi_body,
    grid=(NP // PBLK,),
    in_specs=[
        pl.BlockSpec((PBLK, H), lambda g: (g, 0)),
        pl.BlockSpec((1, 1, PBLK), lambda g: (g, 0, 0)),
        _full((H, 64)),
        _full((1, 64)),
        _full((64, 4)),
        _full((1, 4)),
    ],
    out_specs=_full((NG, 4)),
    out_shape=jax.ShapeDtypeStruct((NG, 4), _f32),
    scratch_shapes=[pltpu.VMEM((NG, H + 1), _f32)],
)


def kernel(x, edge_index, batch, enc_W, enc_b, gcn_W, gcn_b, symp_W,
           dec_W1, dec_b1, dec_W2, dec_b2):
    src = edge_index[0].astype(jnp.int32)
    dst = edge_index[1].astype(jnp.int32)
    epad = EP - E
    src3 = jnp.concatenate([src, jnp.full((epad,), N, jnp.int32)]
                           ).reshape(16, 784, 128)
    dst3 = jnp.concatenate([dst, jnp.full((epad,), N, jnp.int32)]
                           ).reshape(16, 784, 128)
    xp = jnp.pad(x.astype(_f32), ((0, NP - N), (0, 0))).reshape(ROWS, 16)
    b3 = jnp.pad(batch.astype(jnp.int32), (0, NP - N),
                 constant_values=NG).reshape(NP // PBLK, 1, PBLK)

    eye4 = jnp.eye(4, dtype=_f32)
    enc_bd = jnp.einsum("ab,ij->aibj", eye4,
                        enc_W.astype(_f32)).reshape(16, 128)
    gcn_bd = jnp.einsum("ab,lij->laibj", eye4,
                        gcn_W.astype(_f32)).reshape(5, 128, 128)
    symp_bd = jnp.einsum("ab,lij->laibj", eye4,
                         symp_W.astype(_f32)).reshape(5, 128, 128)
    enc_b4 = jnp.tile(enc_b.astype(_f32), 4).reshape(1, 128)
    gcn_b4 = jnp.tile(gcn_b.astype(_f32), (1, 4)).reshape(5, 1, 128)

    deg_p = _sc_deg(dst3).reshape(ROWS, 128)
    h, y, dinv = _tc_pro(xp, deg_p, enc_bd, enc_b4, gcn_bd[0])
    for i in range(5):
        agg_p = _sc_layer(y.reshape(NP, H), src3, dst3).reshape(ROWS, 128)
        if i < 4:
            h, y = _tc_layer(h, y, agg_p, dinv, gcn_b4[i], symp_bd[i],
                             gcn_bd[i + 1])
        else:
            h = _tc_layer_last(h, y, agg_p, dinv, gcn_b4[i], symp_bd[i],
                               gcn_bd[0])
    return _tc_epi(h.reshape(NP, H), b3, dec_W1.astype(_f32),
                   dec_b1.astype(_f32).reshape(1, 64), dec_W2.astype(_f32),
                   dec_b2.astype(_f32).reshape(1, 4))
